# baseline (speedup 1.0000x reference)
import numpy as np

import jax
import jax.numpy as jnp
from jax.experimental import pallas as pl

B = 100
N = 101
E = 128
HID = 512
NODES = B * N
GB = 20
STEPS = B // GB
EPS = 1e-5
F32 = jnp.float32
BF = jnp.bfloat16


def _tri_matrix():
    j = np.arange(N, dtype=np.float64)
    dinv = 1.0 / np.sqrt(j + 1.0)
    a = np.tril(np.ones((N, N))) * (dinv[:, None] * dinv[None, :])
    return jnp.asarray(a, dtype=F32)


def _bn_coefs(s_ref, q_ref, g_ref, b_ref):
    mu = s_ref[...] * (1.0 / NODES)
    var = q_ref[...] * (1.0 / NODES) - mu * mu
    scale = g_ref[...] * jax.lax.rsqrt(var + EPS)
    shift = b_ref[...] - mu * scale
    return scale, shift


def _acc_stats(step, sa, qa, so_ref, qo_ref):
    @pl.when(step == 0)
    def _():
        so_ref[...] = sa
        qo_ref[...] = qa

    @pl.when(step != 0)
    def _():
        so_ref[...] += sa
        qo_ref[...] += qa


def _gcn_tail(i, z, wg_ref, bg_ref, a_ref, y_ref, sa, qa):
    h = jnp.dot(z, wg_ref[...], preferred_element_type=F32)
    m = jnp.dot(a_ref[...], h, preferred_element_type=F32)
    y = z + m + bg_ref[...]
    y_ref[i] = y
    return (sa + jnp.sum(y, axis=0, keepdims=True),
            qa + jnp.sum(y * y, axis=0, keepdims=True))


def _gcn0_body(u_ref, wcat_ref, wg_ref, bg_ref, a_ref, y_ref, so_ref, qo_ref):
    step = pl.program_id(0)
    sa = jnp.zeros((1, E), F32)
    qa = jnp.zeros((1, E), F32)
    for i in range(GB):
        z = jnp.dot(u_ref[i], wcat_ref[...], preferred_element_type=F32)
        sa, qa = _gcn_tail(i, z, wg_ref, bg_ref, a_ref, y_ref, sa, qa)
    _acc_stats(step, sa, qa, so_ref, qo_ref)


def _gcn_bn_body(x_ref, s_ref, q_ref, g_ref, b_ref, wg_ref, bg_ref, a_ref,
                 y_ref, so_ref, qo_ref):
    step = pl.program_id(0)
    scale, shift = _bn_coefs(s_ref, q_ref, g_ref, b_ref)
    sa = jnp.zeros((1, E), F32)
    qa = jnp.zeros((1, E), F32)
    for i in range(GB):
        z = x_ref[i] * scale + shift
        sa, qa = _gcn_tail(i, z, wg_ref, bg_ref, a_ref, y_ref, sa, qa)
    _acc_stats(step, sa, qa, so_ref, qo_ref)


def _ff_body(x_ref, s_ref, q_ref, g_ref, b_ref, w1_ref, b1_ref, w2_ref,
             b2_ref, t_ref, so_ref, qo_ref):
    step = pl.program_id(0)
    scale, shift = _bn_coefs(s_ref, q_ref, g_ref, b_ref)
    sa = jnp.zeros((1, E), F32)
    qa = jnp.zeros((1, E), F32)
    for i in range(GB):
        z = x_ref[i] * scale + shift
        h1 = jnp.maximum(
            jnp.dot(z.astype(BF), w1_ref[...],
                    preferred_element_type=F32) + b1_ref[...], 0.0)
        t = z + jnp.dot(h1.astype(BF), w2_ref[...],
                        preferred_element_type=F32) + b2_ref[...]
        t_ref[i] = t
        sa = sa + jnp.sum(t, axis=0, keepdims=True)
        qa = qa + jnp.sum(t * t, axis=0, keepdims=True)
    _acc_stats(step, sa, qa, so_ref, qo_ref)


def _out_body(x_ref, s_ref, q_ref, g_ref, b_ref, xo_ref, m_ref):
    scale, shift = _bn_coefs(s_ref, q_ref, g_ref, b_ref)
    for i in range(GB):
        z = x_ref[i] * scale + shift
        xo_ref[i] = z
        m_ref[i] = jnp.sum(z, axis=0, keepdims=True) * (1.0 / N)


def _x3_spec(width=E):
    return pl.BlockSpec((GB, N, width), lambda i: (i, 0, 0))


def _const_spec(shape):
    nd = len(shape)
    return pl.BlockSpec(shape, lambda i: (0,) * nd)


def kernel(depot_xy, customer_xy, demand, params):
    z1 = jnp.zeros((B, 1), F32)
    o1 = jnp.ones((B, 1), F32)
    row0 = jnp.concatenate([depot_xy, jnp.zeros((B, 3), F32), o1, z1],
                           axis=1)[:, None, :]
    zc = jnp.zeros((B, N - 1, 1), F32)
    oc = jnp.ones((B, N - 1, 1), F32)
    custp = jnp.concatenate(
        [jnp.zeros((B, N - 1, 2), F32), customer_xy, demand[..., None],
         zc, oc], axis=2)
    u = jnp.concatenate([row0, custp], axis=1)
    wcat = jnp.concatenate(
        [params["Wd"], params["Wi"],
         params["bd"][None, :], params["bi"][None, :]], axis=0)

    a_mat = _tri_matrix()
    x3_out = jax.ShapeDtypeStruct((B, N, E), F32)
    st_out = jax.ShapeDtypeStruct((1, E), F32)
    stats = None
    prev_gb = None
    for li, lp in enumerate(params["layers"]):
        bg = lp["bg"].reshape(1, E)
        if li == 0:
            y3, s1, q1 = pl.pallas_call(
                _gcn0_body,
                grid=(STEPS,),
                in_specs=[_x3_spec(7), _const_spec((7, E)),
                          _const_spec((E, E)), _const_spec((1, E)),
                          _const_spec((N, N))],
                out_specs=[_x3_spec(), _const_spec((1, E)),
                           _const_spec((1, E))],
                out_shape=[x3_out, st_out, st_out],
            )(u, wcat, lp["Wg"], bg, a_mat)
        else:
            s0, q0 = stats
            y3, s1, q1 = pl.pallas_call(
                _gcn_bn_body,
                grid=(STEPS,),
                in_specs=[_x3_spec(), _const_spec((1, E)), _const_spec((1, E)),
                          _const_spec((1, E)), _const_spec((1, E)),
                          _const_spec((E, E)), _const_spec((1, E)),
                          _const_spec((N, N))],
                out_specs=[_x3_spec(), _const_spec((1, E)),
                           _const_spec((1, E))],
                out_shape=[x3_out, st_out, st_out],
            )(x3, s0, q0, prev_gb[0], prev_gb[1], lp["Wg"], bg, a_mat)

        gamma = lp["gamma"].reshape(1, E)
        beta = lp["beta"].reshape(1, E)
        x3, s2, q2 = pl.pallas_call(
            _ff_body,
            grid=(STEPS,),
            in_specs=[_x3_spec(), _const_spec((1, E)), _const_spec((1, E)),
                      _const_spec((1, E)), _const_spec((1, E)),
                      _const_spec((E, HID)), _const_spec((1, HID)),
                      _const_spec((HID, E)), _const_spec((1, E))],
            out_specs=[_x3_spec(), _const_spec((1, E)), _const_spec((1, E))],
            out_shape=[x3_out, st_out, st_out],
        )(y3, s1, q1, gamma, beta, lp["W1"].astype(BF),
          lp["b1"].reshape(1, HID), lp["W2"].astype(BF),
          lp["b2"].reshape(1, E))
        stats = (s2, q2)
        prev_gb = (gamma, beta)

    xf, mf = pl.pallas_call(
        _out_body,
        grid=(STEPS,),
        in_specs=[_x3_spec(), _const_spec((1, E)), _const_spec((1, E)),
                  _const_spec((1, E)), _const_spec((1, E))],
        out_specs=[_x3_spec(), pl.BlockSpec((GB, 1, E), lambda i: (i, 0, 0))],
        out_shape=[x3_out, jax.ShapeDtypeStruct((B, 1, E), F32)],
    )(x3, stats[0], stats[1], prev_gb[0], prev_gb[1])

    return xf, mf.reshape(B, E)


# speedup vs baseline: 1.0604x; 1.0604x over previous
import numpy as np

import jax
import jax.numpy as jnp
from jax.experimental import pallas as pl

B = 100
N = 101
E = 128
HID = 512
NODES = B * N
GB = 20
STEPS = B // GB
EPS = 1e-5
F32 = jnp.float32
BF = jnp.bfloat16


def _tri_matrix():
    j = np.arange(N, dtype=np.float64)
    dinv = 1.0 / np.sqrt(j + 1.0)
    a = np.tril(np.ones((N, N))) * (dinv[:, None] * dinv[None, :])
    return jnp.asarray(a, dtype=F32)


def _bn_coefs(s_ref, q_ref, g_ref, b_ref):
    mu = s_ref[...] * (1.0 / NODES)
    var = q_ref[...] * (1.0 / NODES) - mu * mu
    scale = g_ref[...] * jax.lax.rsqrt(var + EPS)
    shift = b_ref[...] - mu * scale
    return scale, shift


def _acc_stats(step, sa, qa, so_ref, qo_ref):
    @pl.when(step == 0)
    def _():
        so_ref[...] = sa
        qo_ref[...] = qa

    @pl.when(step != 0)
    def _():
        so_ref[...] += sa
        qo_ref[...] += qa


def _gcn_tail(i, z, wg_ref, bg_ref, a_ref, y_ref, sa, qa):
    h = jnp.dot(z, wg_ref[...], preferred_element_type=F32)
    m = jnp.dot(a_ref[...], h, preferred_element_type=F32)
    y = z + m + bg_ref[...]
    y_ref[i] = y
    return (sa + jnp.sum(y, axis=0, keepdims=True),
            qa + jnp.sum(y * y, axis=0, keepdims=True))


def _gcn0_body(u_ref, wcat_ref, wg_ref, bg_ref, a_ref, y_ref, so_ref, qo_ref):
    step = pl.program_id(0)
    sa = jnp.zeros((1, E), F32)
    qa = jnp.zeros((1, E), F32)
    for i in range(GB):
        z = jnp.dot(u_ref[i], wcat_ref[...], preferred_element_type=F32)
        sa, qa = _gcn_tail(i, z, wg_ref, bg_ref, a_ref, y_ref, sa, qa)
    _acc_stats(step, sa, qa, so_ref, qo_ref)


def _gcn_bn_body(x_ref, s_ref, q_ref, g_ref, b_ref, wg_ref, bg_ref, a_ref,
                 y_ref, so_ref, qo_ref):
    step = pl.program_id(0)
    scale, shift = _bn_coefs(s_ref, q_ref, g_ref, b_ref)
    sa = jnp.zeros((1, E), F32)
    qa = jnp.zeros((1, E), F32)
    for i in range(GB):
        z = x_ref[i] * scale + shift
        sa, qa = _gcn_tail(i, z, wg_ref, bg_ref, a_ref, y_ref, sa, qa)
    _acc_stats(step, sa, qa, so_ref, qo_ref)


def _ff_body(x_ref, s_ref, q_ref, g_ref, b_ref, w1_ref, b1_ref, w2_ref,
             b2_ref, t_ref, so_ref, qo_ref):
    step = pl.program_id(0)
    scale, shift = _bn_coefs(s_ref, q_ref, g_ref, b_ref)
    sa = jnp.zeros((1, E), F32)
    qa = jnp.zeros((1, E), F32)
    for i in range(GB):
        z = x_ref[i] * scale + shift
        h1 = jnp.maximum(
            jnp.dot(z, w1_ref[...],
                    preferred_element_type=F32) + b1_ref[...], 0.0)
        t = z + jnp.dot(h1, w2_ref[...],
                        preferred_element_type=F32) + b2_ref[...]
        t_ref[i] = t
        sa = sa + jnp.sum(t, axis=0, keepdims=True)
        qa = qa + jnp.sum(t * t, axis=0, keepdims=True)
    _acc_stats(step, sa, qa, so_ref, qo_ref)


def _out_body(x_ref, s_ref, q_ref, g_ref, b_ref, xo_ref, m_ref):
    scale, shift = _bn_coefs(s_ref, q_ref, g_ref, b_ref)
    for i in range(GB):
        z = x_ref[i] * scale + shift
        xo_ref[i] = z
        m_ref[i] = jnp.sum(z, axis=0, keepdims=True) * (1.0 / N)


def _x3_spec(width=E):
    return pl.BlockSpec((GB, N, width), lambda i: (i, 0, 0))


def _const_spec(shape):
    nd = len(shape)
    return pl.BlockSpec(shape, lambda i: (0,) * nd)


def kernel(depot_xy, customer_xy, demand, params):
    z1 = jnp.zeros((B, 1), F32)
    o1 = jnp.ones((B, 1), F32)
    row0 = jnp.concatenate([depot_xy, jnp.zeros((B, 3), F32), o1, z1],
                           axis=1)[:, None, :]
    zc = jnp.zeros((B, N - 1, 1), F32)
    oc = jnp.ones((B, N - 1, 1), F32)
    custp = jnp.concatenate(
        [jnp.zeros((B, N - 1, 2), F32), customer_xy, demand[..., None],
         zc, oc], axis=2)
    u = jnp.concatenate([row0, custp], axis=1)
    wcat = jnp.concatenate(
        [params["Wd"], params["Wi"],
         params["bd"][None, :], params["bi"][None, :]], axis=0)

    a_mat = _tri_matrix()
    x3_out = jax.ShapeDtypeStruct((B, N, E), F32)
    st_out = jax.ShapeDtypeStruct((1, E), F32)
    stats = None
    prev_gb = None
    for li, lp in enumerate(params["layers"]):
        bg = lp["bg"].reshape(1, E)
        if li == 0:
            y3, s1, q1 = pl.pallas_call(
                _gcn0_body,
                grid=(STEPS,),
                in_specs=[_x3_spec(7), _const_spec((7, E)),
                          _const_spec((E, E)), _const_spec((1, E)),
                          _const_spec((N, N))],
                out_specs=[_x3_spec(), _const_spec((1, E)),
                           _const_spec((1, E))],
                out_shape=[x3_out, st_out, st_out],
            )(u, wcat, lp["Wg"], bg, a_mat)
        else:
            s0, q0 = stats
            y3, s1, q1 = pl.pallas_call(
                _gcn_bn_body,
                grid=(STEPS,),
                in_specs=[_x3_spec(), _const_spec((1, E)), _const_spec((1, E)),
                          _const_spec((1, E)), _const_spec((1, E)),
                          _const_spec((E, E)), _const_spec((1, E)),
                          _const_spec((N, N))],
                out_specs=[_x3_spec(), _const_spec((1, E)),
                           _const_spec((1, E))],
                out_shape=[x3_out, st_out, st_out],
            )(x3, s0, q0, prev_gb[0], prev_gb[1], lp["Wg"], bg, a_mat)

        gamma = lp["gamma"].reshape(1, E)
        beta = lp["beta"].reshape(1, E)
        x3, s2, q2 = pl.pallas_call(
            _ff_body,
            grid=(STEPS,),
            in_specs=[_x3_spec(), _const_spec((1, E)), _const_spec((1, E)),
                      _const_spec((1, E)), _const_spec((1, E)),
                      _const_spec((E, HID)), _const_spec((1, HID)),
                      _const_spec((HID, E)), _const_spec((1, E))],
            out_specs=[_x3_spec(), _const_spec((1, E)), _const_spec((1, E))],
            out_shape=[x3_out, st_out, st_out],
        )(y3, s1, q1, gamma, beta, lp["W1"],
          lp["b1"].reshape(1, HID), lp["W2"],
          lp["b2"].reshape(1, E))
        stats = (s2, q2)
        prev_gb = (gamma, beta)

    xf, mf = pl.pallas_call(
        _out_body,
        grid=(STEPS,),
        in_specs=[_x3_spec(), _const_spec((1, E)), _const_spec((1, E)),
                  _const_spec((1, E)), _const_spec((1, E))],
        out_specs=[_x3_spec(), pl.BlockSpec((GB, 1, E), lambda i: (i, 0, 0))],
        out_shape=[x3_out, jax.ShapeDtypeStruct((B, 1, E), F32)],
    )(x3, stats[0], stats[1], prev_gb[0], prev_gb[1])

    return xf, mf.reshape(B, E)


# speedup vs baseline: 2.0154x; 1.9007x over previous
import numpy as np

import jax
import jax.numpy as jnp
from jax.experimental import pallas as pl

B = 100
N = 101
NP = 104
E = 128
HID = 512
NODES = B * N
ROWS = B * NP
GB = 25
STEPS = B // GB
CH = 5200
CSTEPS = ROWS // CH
EPS = 1e-5
F32 = jnp.float32


def _tri_matrix():
    j = np.arange(NP, dtype=np.float64)
    dinv = 1.0 / np.sqrt(j + 1.0)
    a = np.tril(np.ones((NP, NP))) * (dinv[:, None] * dinv[None, :])
    a[N:, :] = 0.0
    a[:, N:] = 0.0
    return jnp.asarray(a, dtype=F32)


def _row_mask():
    m = (np.arange(ROWS) % NP) < N
    return jnp.asarray(m.astype(np.float32).reshape(ROWS, 1))


def _bn_coefs(s_ref, q_ref, g_ref, b_ref):
    mu = s_ref[...] * (1.0 / NODES)
    var = q_ref[...] * (1.0 / NODES) - mu * mu
    scale = g_ref[...] * jax.lax.rsqrt(var + EPS)
    shift = b_ref[...] - mu * scale
    return scale, shift


def _acc_stats(step, sa, qa, so_ref, qo_ref):
    @pl.when(step == 0)
    def _():
        so_ref[...] = sa
        qo_ref[...] = qa

    @pl.when(step != 0)
    def _():
        so_ref[...] += sa
        qo_ref[...] += qa


def _emb_body(u_ref, wcat_ref, x_ref):
    x_ref[...] = jnp.dot(u_ref[...], wcat_ref[...],
                         preferred_element_type=F32)


def _gcn_tail(i, z, rmask, wg_ref, bg_ref, a_ref, y_ref, sa, qa):
    h = jnp.dot(z, wg_ref[...], preferred_element_type=F32)
    m = jnp.dot(a_ref[...], h, preferred_element_type=F32)
    y = (z + m + bg_ref[...]) * rmask
    y_ref[i] = y
    return (sa + jnp.sum(y, axis=0, keepdims=True),
            qa + jnp.sum(y * y, axis=0, keepdims=True))


def _gcn0_body(x_ref, wg_ref, bg_ref, a_ref, y_ref, so_ref, qo_ref):
    step = pl.program_id(0)
    rmask = (jax.lax.broadcasted_iota(jnp.int32, (NP, 1), 0) < N).astype(F32)
    sa = jnp.zeros((1, E), F32)
    qa = jnp.zeros((1, E), F32)
    for i in range(GB):
        sa, qa = _gcn_tail(i, x_ref[i], rmask, wg_ref, bg_ref, a_ref, y_ref,
                           sa, qa)
    _acc_stats(step, sa, qa, so_ref, qo_ref)


def _gcn_bn_body(x_ref, s_ref, q_ref, g_ref, b_ref, wg_ref, bg_ref, a_ref,
                 y_ref, so_ref, qo_ref):
    step = pl.program_id(0)
    rmask = (jax.lax.broadcasted_iota(jnp.int32, (NP, 1), 0) < N).astype(F32)
    scale, shift = _bn_coefs(s_ref, q_ref, g_ref, b_ref)
    sa = jnp.zeros((1, E), F32)
    qa = jnp.zeros((1, E), F32)
    for i in range(GB):
        z = x_ref[i] * scale + shift
        sa, qa = _gcn_tail(i, z, rmask, wg_ref, bg_ref, a_ref, y_ref, sa, qa)
    _acc_stats(step, sa, qa, so_ref, qo_ref)


def _ff_body(x_ref, m_ref, s_ref, q_ref, g_ref, b_ref, w1_ref, b1_ref,
             w2_ref, b2_ref, t_ref, so_ref, qo_ref):
    step = pl.program_id(0)
    scale, shift = _bn_coefs(s_ref, q_ref, g_ref, b_ref)
    z = x_ref[...] * scale + shift
    h1 = jnp.maximum(
        jnp.dot(z, w1_ref[...], preferred_element_type=F32) + b1_ref[...],
        0.0)
    t = (z + jnp.dot(h1, w2_ref[...], preferred_element_type=F32)
         + b2_ref[...]) * m_ref[...]
    t_ref[...] = t
    sa = jnp.sum(t, axis=0, keepdims=True)
    qa = jnp.sum(t * t, axis=0, keepdims=True)
    _acc_stats(step, sa, qa, so_ref, qo_ref)


def _out_body(x_ref, s_ref, q_ref, g_ref, b_ref, xo_ref, m_ref):
    scale, shift = _bn_coefs(s_ref, q_ref, g_ref, b_ref)
    for i in range(GB):
        z = x_ref[i] * scale + shift
        zs = z[:N, :]
        xo_ref[i] = zs
        m_ref[i] = jnp.sum(zs, axis=0, keepdims=True) * (1.0 / N)


def _x3_spec(width=E, rows=NP):
    return pl.BlockSpec((GB, rows, width), lambda i: (i, 0, 0))


def _x2_spec(width=E):
    return pl.BlockSpec((CH, width), lambda i: (i, 0))


def _const_spec(shape):
    nd = len(shape)
    return pl.BlockSpec(shape, lambda i: (0,) * nd)


def kernel(depot_xy, customer_xy, demand, params):
    z1 = jnp.zeros((B, 1), F32)
    o1 = jnp.ones((B, 1), F32)
    row0 = jnp.concatenate([depot_xy, jnp.zeros((B, 3), F32), o1, z1],
                           axis=1)[:, None, :]
    zc = jnp.zeros((B, N - 1, 1), F32)
    oc = jnp.ones((B, N - 1, 1), F32)
    custp = jnp.concatenate(
        [jnp.zeros((B, N - 1, 2), F32), customer_xy, demand[..., None],
         zc, oc], axis=2)
    pad = jnp.zeros((B, NP - N, 7), F32)
    u2 = jnp.concatenate([row0, custp, pad], axis=1).reshape(ROWS, 7)
    wcat = jnp.concatenate(
        [params["Wd"], params["Wi"],
         params["bd"][None, :], params["bi"][None, :]], axis=0)

    a_mat = _tri_matrix()
    rmask2 = _row_mask()

    x2_out = jax.ShapeDtypeStruct((ROWS, E), F32)
    x3_out = jax.ShapeDtypeStruct((B, NP, E), F32)
    st_out = jax.ShapeDtypeStruct((1, E), F32)

    x2 = pl.pallas_call(
        _emb_body,
        grid=(CSTEPS,),
        in_specs=[_x2_spec(7), _const_spec((7, E))],
        out_specs=_x2_spec(),
        out_shape=x2_out,
    )(u2, wcat)

    stats = None
    prev_gb = None
    for li, lp in enumerate(params["layers"]):
        bg = lp["bg"].reshape(1, E)
        x3 = x2.reshape(B, NP, E)
        if li == 0:
            y3, s1, q1 = pl.pallas_call(
                _gcn0_body,
                grid=(STEPS,),
                in_specs=[_x3_spec(), _const_spec((E, E)),
                          _const_spec((1, E)), _const_spec((NP, NP))],
                out_specs=[_x3_spec(), _const_spec((1, E)),
                           _const_spec((1, E))],
                out_shape=[x3_out, st_out, st_out],
            )(x3, lp["Wg"], bg, a_mat)
        else:
            s0, q0 = stats
            y3, s1, q1 = pl.pallas_call(
                _gcn_bn_body,
                grid=(STEPS,),
                in_specs=[_x3_spec(), _const_spec((1, E)), _const_spec((1, E)),
                          _const_spec((1, E)), _const_spec((1, E)),
                          _const_spec((E, E)), _const_spec((1, E)),
                          _const_spec((NP, NP))],
                out_specs=[_x3_spec(), _const_spec((1, E)),
                           _const_spec((1, E))],
                out_shape=[x3_out, st_out, st_out],
            )(x3, s0, q0, prev_gb[0], prev_gb[1], lp["Wg"], bg, a_mat)

        gamma = lp["gamma"].reshape(1, E)
        beta = lp["beta"].reshape(1, E)
        x2, s2, q2 = pl.pallas_call(
            _ff_body,
            grid=(CSTEPS,),
            in_specs=[_x2_spec(), pl.BlockSpec((CH, 1), lambda i: (i, 0)),
                      _const_spec((1, E)), _const_spec((1, E)),
                      _const_spec((1, E)), _const_spec((1, E)),
                      _const_spec((E, HID)), _const_spec((1, HID)),
                      _const_spec((HID, E)), _const_spec((1, E))],
            out_specs=[_x2_spec(), _const_spec((1, E)), _const_spec((1, E))],
            out_shape=[x2_out, st_out, st_out],
        )(y3.reshape(ROWS, E), rmask2, s1, q1, gamma, beta, lp["W1"],
          lp["b1"].reshape(1, HID), lp["W2"], lp["b2"].reshape(1, E))
        stats = (s2, q2)
        prev_gb = (gamma, beta)

    xf, mf = pl.pallas_call(
        _out_body,
        grid=(STEPS,),
        in_specs=[_x3_spec(), _const_spec((1, E)), _const_spec((1, E)),
                  _const_spec((1, E)), _const_spec((1, E))],
        out_specs=[pl.BlockSpec((GB, N, E), lambda i: (i, 0, 0)),
                   pl.BlockSpec((GB, 1, E), lambda i: (i, 0, 0))],
        out_shape=[jax.ShapeDtypeStruct((B, N, E), F32),
                   jax.ShapeDtypeStruct((B, 1, E), F32)],
    )(x2.reshape(B, NP, E), stats[0], stats[1], prev_gb[0], prev_gb[1])

    return xf, mf.reshape(B, E)


# speedup vs baseline: 2.1214x; 1.0526x over previous
import numpy as np

import jax
import jax.numpy as jnp
from jax.experimental import pallas as pl

B = 100
N = 101
NP = 104
E = 128
HID = 512
NODES = B * N
ROWS = B * NP
GB = 50
STEPS = B // GB
CH = 5200
CSTEPS = ROWS // CH
EPS = 1e-5
F32 = jnp.float32


def _tri_matrix():
    j = np.arange(NP, dtype=np.float64)
    dinv = 1.0 / np.sqrt(j + 1.0)
    a = np.tril(np.ones((NP, NP))) * (dinv[:, None] * dinv[None, :])
    a[N:, :] = 0.0
    a[:, N:] = 0.0
    return jnp.asarray(a, dtype=F32)


def _row_mask():
    m = (np.arange(ROWS) % NP) < N
    return jnp.asarray(m.astype(np.float32).reshape(ROWS, 1))


def _bn_coefs(s_ref, q_ref, g_ref, b_ref):
    mu = s_ref[...] * (1.0 / NODES)
    var = q_ref[...] * (1.0 / NODES) - mu * mu
    scale = g_ref[...] * jax.lax.rsqrt(var + EPS)
    shift = b_ref[...] - mu * scale
    return scale, shift


def _acc_stats(step, sa, qa, so_ref, qo_ref):
    @pl.when(step == 0)
    def _():
        so_ref[...] = sa
        qo_ref[...] = qa

    @pl.when(step != 0)
    def _():
        so_ref[...] += sa
        qo_ref[...] += qa


def _emb_body(u_ref, wcat_ref, x_ref):
    x_ref[...] = jnp.dot(u_ref[...], wcat_ref[...],
                         preferred_element_type=F32)


def _gcn_tail(i, z, rmask, wg_ref, bg_ref, a_ref, y_ref, sa, qa):
    h = jnp.dot(z, wg_ref[...], preferred_element_type=F32)
    m = jnp.dot(a_ref[...], h, preferred_element_type=F32)
    y = (z + m + bg_ref[...]) * rmask
    y_ref[i] = y
    return (sa + jnp.sum(y, axis=0, keepdims=True),
            qa + jnp.sum(y * y, axis=0, keepdims=True))


def _gcn0_body(x_ref, wg_ref, bg_ref, a_ref, y_ref, so_ref, qo_ref):
    step = pl.program_id(0)
    rmask = (jax.lax.broadcasted_iota(jnp.int32, (NP, 1), 0) < N).astype(F32)
    sa = jnp.zeros((1, E), F32)
    qa = jnp.zeros((1, E), F32)
    for i in range(GB):
        sa, qa = _gcn_tail(i, x_ref[i], rmask, wg_ref, bg_ref, a_ref, y_ref,
                           sa, qa)
    _acc_stats(step, sa, qa, so_ref, qo_ref)


def _gcn_bn_body(x_ref, s_ref, q_ref, g_ref, b_ref, wg_ref, bg_ref, a_ref,
                 y_ref, so_ref, qo_ref):
    step = pl.program_id(0)
    rmask = (jax.lax.broadcasted_iota(jnp.int32, (NP, 1), 0) < N).astype(F32)
    scale, shift = _bn_coefs(s_ref, q_ref, g_ref, b_ref)
    sa = jnp.zeros((1, E), F32)
    qa = jnp.zeros((1, E), F32)
    for i in range(GB):
        z = x_ref[i] * scale + shift
        sa, qa = _gcn_tail(i, z, rmask, wg_ref, bg_ref, a_ref, y_ref, sa, qa)
    _acc_stats(step, sa, qa, so_ref, qo_ref)


def _ff_body(x_ref, m_ref, s_ref, q_ref, g_ref, b_ref, w1_ref, b1_ref,
             w2_ref, b2_ref, t_ref, so_ref, qo_ref):
    step = pl.program_id(0)
    scale, shift = _bn_coefs(s_ref, q_ref, g_ref, b_ref)
    z = x_ref[...] * scale + shift
    h1 = jnp.maximum(
        jnp.dot(z, w1_ref[...], preferred_element_type=F32) + b1_ref[...],
        0.0)
    t = (z + jnp.dot(h1, w2_ref[...], preferred_element_type=F32)
         + b2_ref[...]) * m_ref[...]
    t_ref[...] = t
    sa = jnp.sum(t, axis=0, keepdims=True)
    qa = jnp.sum(t * t, axis=0, keepdims=True)
    _acc_stats(step, sa, qa, so_ref, qo_ref)


def _out_body(x_ref, s_ref, q_ref, g_ref, b_ref, xo_ref, m_ref):
    scale, shift = _bn_coefs(s_ref, q_ref, g_ref, b_ref)
    for i in range(GB):
        z = x_ref[i] * scale + shift
        zs = z[:N, :]
        xo_ref[i] = zs
        m_ref[i] = jnp.sum(zs, axis=0, keepdims=True) * (1.0 / N)


def _x3_spec(width=E, rows=NP):
    return pl.BlockSpec((GB, rows, width), lambda i: (i, 0, 0))


def _x2_spec(width=E):
    return pl.BlockSpec((CH, width), lambda i: (i, 0))


def _const_spec(shape):
    nd = len(shape)
    return pl.BlockSpec(shape, lambda i: (0,) * nd)


def kernel(depot_xy, customer_xy, demand, params):
    z1 = jnp.zeros((B, 1), F32)
    o1 = jnp.ones((B, 1), F32)
    row0 = jnp.concatenate([depot_xy, jnp.zeros((B, 3), F32), o1, z1],
                           axis=1)[:, None, :]
    zc = jnp.zeros((B, N - 1, 1), F32)
    oc = jnp.ones((B, N - 1, 1), F32)
    custp = jnp.concatenate(
        [jnp.zeros((B, N - 1, 2), F32), customer_xy, demand[..., None],
         zc, oc], axis=2)
    pad = jnp.zeros((B, NP - N, 7), F32)
    u2 = jnp.concatenate([row0, custp, pad], axis=1).reshape(ROWS, 7)
    wcat = jnp.concatenate(
        [params["Wd"], params["Wi"],
         params["bd"][None, :], params["bi"][None, :]], axis=0)

    a_mat = _tri_matrix()
    rmask2 = _row_mask()

    x2_out = jax.ShapeDtypeStruct((ROWS, E), F32)
    x3_out = jax.ShapeDtypeStruct((B, NP, E), F32)
    st_out = jax.ShapeDtypeStruct((1, E), F32)

    x2 = pl.pallas_call(
        _emb_body,
        grid=(CSTEPS,),
        in_specs=[_x2_spec(7), _const_spec((7, E))],
        out_specs=_x2_spec(),
        out_shape=x2_out,
    )(u2, wcat)

    stats = None
    prev_gb = None
    for li, lp in enumerate(params["layers"]):
        bg = lp["bg"].reshape(1, E)
        x3 = x2.reshape(B, NP, E)
        if li == 0:
            y3, s1, q1 = pl.pallas_call(
                _gcn0_body,
                grid=(STEPS,),
                in_specs=[_x3_spec(), _const_spec((E, E)),
                          _const_spec((1, E)), _const_spec((NP, NP))],
                out_specs=[_x3_spec(), _const_spec((1, E)),
                           _const_spec((1, E))],
                out_shape=[x3_out, st_out, st_out],
            )(x3, lp["Wg"], bg, a_mat)
        else:
            s0, q0 = stats
            y3, s1, q1 = pl.pallas_call(
                _gcn_bn_body,
                grid=(STEPS,),
                in_specs=[_x3_spec(), _const_spec((1, E)), _const_spec((1, E)),
                          _const_spec((1, E)), _const_spec((1, E)),
                          _const_spec((E, E)), _const_spec((1, E)),
                          _const_spec((NP, NP))],
                out_specs=[_x3_spec(), _const_spec((1, E)),
                           _const_spec((1, E))],
                out_shape=[x3_out, st_out, st_out],
            )(x3, s0, q0, prev_gb[0], prev_gb[1], lp["Wg"], bg, a_mat)

        gamma = lp["gamma"].reshape(1, E)
        beta = lp["beta"].reshape(1, E)
        x2, s2, q2 = pl.pallas_call(
            _ff_body,
            grid=(CSTEPS,),
            in_specs=[_x2_spec(), pl.BlockSpec((CH, 1), lambda i: (i, 0)),
                      _const_spec((1, E)), _const_spec((1, E)),
                      _const_spec((1, E)), _const_spec((1, E)),
                      _const_spec((E, HID)), _const_spec((1, HID)),
                      _const_spec((HID, E)), _const_spec((1, E))],
            out_specs=[_x2_spec(), _const_spec((1, E)), _const_spec((1, E))],
            out_shape=[x2_out, st_out, st_out],
        )(y3.reshape(ROWS, E), rmask2, s1, q1, gamma, beta, lp["W1"],
          lp["b1"].reshape(1, HID), lp["W2"], lp["b2"].reshape(1, E))
        stats = (s2, q2)
        prev_gb = (gamma, beta)

    xf, mf = pl.pallas_call(
        _out_body,
        grid=(STEPS,),
        in_specs=[_x3_spec(), _const_spec((1, E)), _const_spec((1, E)),
                  _const_spec((1, E)), _const_spec((1, E))],
        out_specs=[pl.BlockSpec((GB, N, E), lambda i: (i, 0, 0)),
                   pl.BlockSpec((GB, 1, E), lambda i: (i, 0, 0))],
        out_shape=[jax.ShapeDtypeStruct((B, N, E), F32),
                   jax.ShapeDtypeStruct((B, 1, E), F32)],
    )(x2.reshape(B, NP, E), stats[0], stats[1], prev_gb[0], prev_gb[1])

    return xf, mf.reshape(B, E)
